# baseline (speedup 1.0000x reference)
import functools

import jax
import jax.numpy as jnp
from jax import lax
from jax.experimental import pallas as pl
from jax.experimental.pallas import tpu as pltpu
from jax.experimental.pallas import tpu_sc as plsc

_NC = 2
_NS = 16
_NW = _NC * _NS
_LANES = 16


def _film_body(od, nf_ref, cond_ref, Wc_ref, bc_ref, Wf_ref, bf_ref, tbl_ref):
    nf = nf_ref[0]
    x = lax.dot_general(nf, Wf_ref[...], (((1,), (1,)), ((), ())),
                        preferred_element_type=jnp.float32)
    x = x + bf_ref[...]
    mu = jnp.mean(x, axis=1, keepdims=True)
    xc = x - mu
    var = jnp.mean(xc * xc, axis=1, keepdims=True)
    xn = xc / jnp.sqrt(var + 1e-5)
    gb = lax.dot_general(cond_ref[0], Wc_ref[...], (((1,), (1,)), ((), ())),
                         preferred_element_type=jnp.float32)
    gb = gb + bc_ref[...]
    gamma = gb[:, :od] + 1.0
    beta = gb[:, od:]
    tbl_ref[...] = jnp.maximum(gamma * xn + beta, 0.0)


def _film_call(node_feats, cond_feats, W_cond, b_cond, W_film, b_film):
    B, N, D = node_feats.shape
    OD = W_film.shape[0]
    CD = W_cond.shape[1]
    return pl.pallas_call(
        functools.partial(_film_body, OD),
        grid=(B,),
        in_specs=[
            pl.BlockSpec((1, N, D), lambda b: (b, 0, 0)),
            pl.BlockSpec((1, 1, CD), lambda b: (b, 0, 0)),
            pl.BlockSpec((2 * OD, CD), lambda b: (0, 0)),
            pl.BlockSpec((1, 2 * OD), lambda b: (0, 0)),
            pl.BlockSpec((OD, D), lambda b: (0, 0)),
            pl.BlockSpec((1, OD), lambda b: (0, 0)),
        ],
        out_specs=pl.BlockSpec((N, OD), lambda b: (b, 0)),
        out_shape=jax.ShapeDtypeStruct((B * N, OD), jnp.float32),
    )(node_feats, cond_feats, W_cond, b_cond.reshape(1, 2 * OD), W_film,
      b_film.reshape(1, OD))


def _make_sc_gather(n_nodes, OD, K, E):
    CE = 128
    CN = CE // K
    NCH = OD // _LANES
    n_chunks = E // CE
    assert n_chunks * CE == E
    BASE = n_chunks // _NW
    assert BASE % 2 == 0
    REM = n_chunks - _NW * BASE
    assert REM % 2 == 0
    NXW = REM // 2
    mesh = plsc.VectorSubcoreMesh(core_axis_name="c", subcore_axis_name="s")

    @functools.partial(
        pl.kernel,
        out_type=jax.ShapeDtypeStruct((n_nodes, OD), jnp.float32),
        mesh=mesh,
        scratch_types=[
            pltpu.VMEM_SHARED((n_nodes, OD), jnp.float32),
            pltpu.VMEM((3, CE), jnp.int32),
            pltpu.VMEM((3, CE), jnp.float32),
            pltpu.VMEM((3, CE), jnp.float32),
            pltpu.VMEM((2, CE, OD), jnp.float32),
            pltpu.VMEM((2, 2 * CN, OD), jnp.float32),
            pltpu.SemaphoreType.DMA((3,)),
            pltpu.SemaphoreType.DMA((2,)),
            pltpu.SemaphoreType.DMA((2,)),
        ],
    )
    def sc_gather(tbl_hbm, idx_hbm, w_hbm, p_hbm, out_hbm, tbl_sh, idx_r,
                  w_r, p_r, rows_v, out_r, sem_in, sem_g, sem_out):
        sid = lax.axis_index("s")
        wid = sid * _NC + lax.axis_index("c")
        start = wid * BASE + 2 * jnp.minimum(wid, NXW)
        ebase = start * CE
        has_extra = wid < NXW
        n_mine = BASE + jnp.where(has_extra, 2, 0)

        shard = (n_nodes // (8 * _NS)) * 8
        r0 = sid * shard
        nlast = n_nodes - (_NS - 1) * shard

        @pl.when(sid < _NS - 1)
        def _():
            pltpu.sync_copy(tbl_hbm.at[pl.ds(r0, shard)],
                            tbl_sh.at[pl.ds(r0, shard)])

        @pl.when(sid == _NS - 1)
        def _():
            pltpu.sync_copy(tbl_hbm.at[pl.ds(r0, nlast)],
                            tbl_sh.at[pl.ds(r0, nlast)])

        plsc.subcore_barrier()

        lane_splat = [jnp.full((_LANES,), j, jnp.int32) for j in range(_LANES)]

        def issue_in(ci, s):
            off = ebase + ci * CE
            pltpu.async_copy(idx_hbm.at[pl.ds(off, CE)], idx_r.at[s],
                             sem_in.at[s])
            pltpu.async_copy(w_hbm.at[pl.ds(off, CE)], w_r.at[s],
                             sem_in.at[s])
            pltpu.async_copy(p_hbm.at[pl.ds(off, CE)], p_r.at[s],
                             sem_in.at[s])

        def wait_in(s):
            pltpu.make_async_copy(idx_hbm.at[pl.ds(0, CE)], idx_r.at[s],
                                  sem_in.at[s]).wait()
            pltpu.make_async_copy(w_hbm.at[pl.ds(0, CE)], w_r.at[s],
                                  sem_in.at[s]).wait()
            pltpu.make_async_copy(p_hbm.at[pl.ds(0, CE)], p_r.at[s],
                                  sem_in.at[s]).wait()

        def issue_g(ci, rslot):
            pltpu.async_copy(tbl_sh.at[idx_r.at[lax.rem(ci, 3)]],
                             rows_v.at[rslot], sem_g.at[rslot])

        def wait_g(rslot):
            pltpu.make_async_copy(tbl_sh.at[idx_r.at[0]], rows_v.at[rslot],
                                  sem_g.at[rslot]).wait()

        def compute(ci, rslot, oslot, opos):
            s_in = lax.rem(ci, 3)
            for q in range(CN):
                acc = [jnp.zeros((_LANES,), jnp.float32) for _ in range(NCH)]
                for g in range(K // _LANES):
                    off = q * K + g * _LANES
                    ew = (w_r[s_in, pl.ds(off, _LANES)] *
                          p_r[s_in, pl.ds(off, _LANES)])
                    for jj in range(_LANES):
                        e = q * K + g * _LANES + jj
                        wb = ew.at[lane_splat[jj]].get(
                            mode='promise_in_bounds')
                        for c in range(NCH):
                            r = rows_v[rslot, e, pl.ds(c * _LANES, _LANES)]
                            acc[c] = acc[c] + wb * r
                row = opos * CN + q
                for c in range(NCH):
                    out_r[oslot, row, pl.ds(c * _LANES, _LANES)] = (
                        jnp.maximum(acc[c], 0.0))

        issue_in(0, 0)
        issue_in(1, 1)
        issue_in(2, 2)
        wait_in(0)
        issue_g(0, 0)

        def pbody(i2, carry):
            oslot = lax.rem(i2, 2)
            for k in range(2):
                j = 2 * i2 + k
                wait_g(k)

                @pl.when(j + 1 < n_mine)
                def _():
                    wait_in(lax.rem(j + 1, 3))
                    issue_g(j + 1, 1 - k)

                if k == 0:
                    @pl.when(i2 >= 2)
                    def _():
                        pltpu.make_async_copy(
                            out_r.at[oslot],
                            out_hbm.at[pl.ds(start * CN, 2 * CN)],
                            sem_out.at[oslot]).wait()

                compute(j, k, oslot, k)

                @pl.when(j + 3 < n_mine)
                def _():
                    issue_in(j + 3, lax.rem(j, 3))

            pltpu.async_copy(out_r.at[oslot],
                             out_hbm.at[pl.ds((start + 2 * i2) * CN, 2 * CN)],
                             sem_out.at[oslot])
            return carry

        lax.fori_loop(0, n_mine // 2, pbody, 0)

        for s in range(2):
            pltpu.make_async_copy(out_r.at[s],
                                  out_hbm.at[pl.ds(start * CN, 2 * CN)],
                                  sem_out.at[s]).wait()

    return sc_gather


def kernel(node_feats, cond_feats, weights, params, coords_j, W_cond, b_cond,
           W_film, b_film):
    B, N, D = node_feats.shape
    K = weights.shape[2]
    OD = W_film.shape[0]
    E = B * N * K

    tbl = _film_call(node_feats, cond_feats, W_cond, b_cond, W_film, b_film)
    sc = _make_sc_gather(B * N, OD, K, E)
    idx = (coords_j if coords_j.dtype == jnp.int32
           else coords_j.astype(jnp.int32))
    out = sc(tbl, idx, weights.reshape(E), params.reshape(E))
    return out.reshape(B, N, OD)


# speedup vs baseline: 1.4243x; 1.4243x over previous
import functools

import jax
import jax.numpy as jnp
from jax import lax
from jax.experimental import pallas as pl
from jax.experimental.pallas import tpu as pltpu
from jax.experimental.pallas import tpu_sc as plsc

_NC = 2
_NS = 16
_NW = _NC * _NS
_LANES = 16


def _film_body(od, n_per, nf_ref, cond_ref, Wc_ref, bc_ref, Wf_ref, bf_ref,
               tbl_ref):
    bpg, N, D = nf_ref.shape
    nf = nf_ref[...].reshape(bpg * N, D)
    x = lax.dot_general(nf, Wf_ref[...], (((1,), (1,)), ((), ())),
                        preferred_element_type=jnp.float32)
    x = x + bf_ref[...]
    mu = jnp.mean(x, axis=1, keepdims=True)
    xc = x - mu
    var = jnp.mean(xc * xc, axis=1, keepdims=True)
    xn = xc / jnp.sqrt(var + 1e-5)
    cond = cond_ref[...].reshape(bpg, cond_ref.shape[2])
    gb = lax.dot_general(cond, Wc_ref[...], (((1,), (1,)), ((), ())),
                         preferred_element_type=jnp.float32)
    gb = gb + bc_ref[...]
    gamma = (gb[:, :od] + 1.0)[:, None, :]
    beta = gb[:, od:][:, None, :]
    gfull = jnp.broadcast_to(gamma, (bpg, N, od)).reshape(bpg * N, od)
    bfull = jnp.broadcast_to(beta, (bpg, N, od)).reshape(bpg * N, od)
    tbl_ref[...] = jnp.maximum(gfull * xn + bfull, 0.0)


def _film_call(node_feats, cond_feats, W_cond, b_cond, W_film, b_film):
    B, N, D = node_feats.shape
    OD = W_film.shape[0]
    CD = W_cond.shape[1]
    G = 2 if B % 2 == 0 else 1
    BPG = B // G
    return pl.pallas_call(
        functools.partial(_film_body, OD, N),
        grid=(G,),
        in_specs=[
            pl.BlockSpec((BPG, N, D), lambda b: (b, 0, 0)),
            pl.BlockSpec((BPG, 1, CD), lambda b: (b, 0, 0)),
            pl.BlockSpec((2 * OD, CD), lambda b: (0, 0)),
            pl.BlockSpec((1, 2 * OD), lambda b: (0, 0)),
            pl.BlockSpec((OD, D), lambda b: (0, 0)),
            pl.BlockSpec((1, OD), lambda b: (0, 0)),
        ],
        out_specs=pl.BlockSpec((BPG * N, OD), lambda b: (b, 0)),
        out_shape=jax.ShapeDtypeStruct((B * N, OD), jnp.float32),
    )(node_feats, cond_feats, W_cond, b_cond.reshape(1, 2 * OD), W_film,
      b_film.reshape(1, OD))


def _make_sc_gather(n_nodes, OD, K, E):
    CE = 128
    CN = CE // K
    NCH = OD // _LANES
    n_chunks = E // CE
    assert n_chunks * CE == E
    BASE = n_chunks // _NW
    assert BASE % 3 == 0
    assert BASE % 2 == 0
    REM = n_chunks - _NW * BASE
    assert REM % 2 == 0
    NXW = REM // 2
    CAP = BASE + (2 if REM else 0)
    mesh = plsc.VectorSubcoreMesh(core_axis_name="c", subcore_axis_name="s")

    @functools.partial(
        pl.kernel,
        out_type=jax.ShapeDtypeStruct((n_nodes, OD), jnp.float32),
        mesh=mesh,
        scratch_types=[
            pltpu.VMEM((CAP * CE,), jnp.int32),
            pltpu.VMEM((CAP * CE,), jnp.float32),
            pltpu.VMEM((CAP * CE,), jnp.float32),
            pltpu.VMEM((3, CE, OD), jnp.float32),
            pltpu.VMEM((CAP * CN, OD), jnp.float32),
            pltpu.SemaphoreType.DMA((3,)),
        ],
    )
    def sc_gather(tbl_hbm, idx_hbm, w_hbm, p_hbm, out_hbm, idx_v, w_v, p_v,
                  rows_v, out_v, sem):
        wid = lax.axis_index("s") * _NC + lax.axis_index("c")
        start = wid * BASE + 2 * jnp.minimum(wid, NXW)
        ebase = start * CE
        has_extra = wid < NXW
        pltpu.sync_copy(idx_hbm.at[pl.ds(ebase, BASE * CE)],
                        idx_v.at[pl.ds(0, BASE * CE)])
        pltpu.sync_copy(w_hbm.at[pl.ds(ebase, BASE * CE)],
                        w_v.at[pl.ds(0, BASE * CE)])
        pltpu.sync_copy(p_hbm.at[pl.ds(ebase, BASE * CE)],
                        p_v.at[pl.ds(0, BASE * CE)])

        @pl.when(has_extra)
        def _():
            eb2 = ebase + BASE * CE
            pltpu.sync_copy(idx_hbm.at[pl.ds(eb2, 2 * CE)],
                            idx_v.at[pl.ds(BASE * CE, 2 * CE)])
            pltpu.sync_copy(w_hbm.at[pl.ds(eb2, 2 * CE)],
                            w_v.at[pl.ds(BASE * CE, 2 * CE)])
            pltpu.sync_copy(p_hbm.at[pl.ds(eb2, 2 * CE)],
                            p_v.at[pl.ds(BASE * CE, 2 * CE)])

        lane_splat = [jnp.full((_LANES,), j, jnp.int32) for j in range(_LANES)]

        def issue(ci, slot):
            pltpu.async_copy(tbl_hbm.at[idx_v.at[pl.ds(ci * CE, CE)]],
                             rows_v.at[slot], sem.at[slot])

        def wait(slot):
            pltpu.make_async_copy(tbl_hbm.at[idx_v.at[pl.ds(0, CE)]],
                                  rows_v.at[slot], sem.at[slot]).wait()

        def compute(ci, slot):
            for q in range(CN):
                acc = [jnp.zeros((_LANES,), jnp.float32) for _ in range(NCH)]
                for g in range(K // _LANES):
                    off = ci * CE + q * K + g * _LANES
                    ew = (w_v[pl.ds(off, _LANES)] * p_v[pl.ds(off, _LANES)])
                    for jj in range(_LANES):
                        e = q * K + g * _LANES + jj
                        wb = ew.at[lane_splat[jj]].get(
                            mode='promise_in_bounds')
                        for c in range(NCH):
                            r = rows_v[slot, e, pl.ds(c * _LANES, _LANES)]
                            acc[c] = acc[c] + wb * r
                row = ci * CN + q
                for c in range(NCH):
                    out_v[row, pl.ds(c * _LANES, _LANES)] = jnp.maximum(
                        acc[c], 0.0)

        n_mine = BASE + jnp.where(has_extra, 2, 0)
        issue(0, 0)
        issue(1, 1)

        def gbody(i, carry):
            slot = lax.rem(i, 3)
            wait(slot)
            compute(i, slot)
            nci = i + 2

            @pl.when(nci < n_mine)
            def _():
                issue(nci, lax.rem(nci, 3))
            return carry

        lax.fori_loop(0, n_mine, gbody, 0)

        pltpu.sync_copy(out_v.at[pl.ds(0, BASE * CN)],
                        out_hbm.at[pl.ds(start * CN, BASE * CN)])

        @pl.when(has_extra)
        def _():
            pltpu.sync_copy(
                out_v.at[pl.ds(BASE * CN, 2 * CN)],
                out_hbm.at[pl.ds(start * CN + BASE * CN, 2 * CN)])

    return sc_gather


def kernel(node_feats, cond_feats, weights, params, coords_j, W_cond, b_cond,
           W_film, b_film):
    B, N, D = node_feats.shape
    K = weights.shape[2]
    OD = W_film.shape[0]
    E = B * N * K

    tbl = _film_call(node_feats, cond_feats, W_cond, b_cond, W_film, b_film)
    sc = _make_sc_gather(B * N, OD, K, E)
    idx = (coords_j if coords_j.dtype == jnp.int32
           else coords_j.astype(jnp.int32))
    out = sc(tbl, idx, weights.reshape(E), params.reshape(E))
    return out.reshape(B, N, OD)


# speedup vs baseline: 1.5742x; 1.1052x over previous
import functools

import jax
import jax.numpy as jnp
from jax import lax
from jax.experimental import pallas as pl
from jax.experimental.pallas import tpu as pltpu
from jax.experimental.pallas import tpu_sc as plsc

_NC = 2
_NS = 16
_NW = _NC * _NS
_LANES = 16


def _film_body(od, n_per, nf_ref, cond_ref, Wc_ref, bc_ref, Wf_ref, bf_ref,
               tbl_ref):
    bpg, N, D = nf_ref.shape
    nf = nf_ref[...].reshape(bpg * N, D)
    x = lax.dot_general(nf, Wf_ref[...], (((1,), (1,)), ((), ())),
                        preferred_element_type=jnp.float32)
    x = x + bf_ref[...]
    mu = jnp.mean(x, axis=1, keepdims=True)
    xc = x - mu
    var = jnp.mean(xc * xc, axis=1, keepdims=True)
    xn = xc / jnp.sqrt(var + 1e-5)
    cond = cond_ref[...].reshape(bpg, cond_ref.shape[2])
    gb = lax.dot_general(cond, Wc_ref[...], (((1,), (1,)), ((), ())),
                         preferred_element_type=jnp.float32)
    gb = gb + bc_ref[...]
    gamma = (gb[:, :od] + 1.0)[:, None, :]
    beta = gb[:, od:][:, None, :]
    gfull = jnp.broadcast_to(gamma, (bpg, N, od)).reshape(bpg * N, od)
    bfull = jnp.broadcast_to(beta, (bpg, N, od)).reshape(bpg * N, od)
    tbl_ref[...] = jnp.maximum(gfull * xn + bfull, 0.0)


def _film_call(node_feats, cond_feats, W_cond, b_cond, W_film, b_film):
    B, N, D = node_feats.shape
    OD = W_film.shape[0]
    CD = W_cond.shape[1]
    G = 2 if B % 2 == 0 else 1
    BPG = B // G
    return pl.pallas_call(
        functools.partial(_film_body, OD, N),
        grid=(G,),
        in_specs=[
            pl.BlockSpec((BPG, N, D), lambda b: (b, 0, 0)),
            pl.BlockSpec((BPG, 1, CD), lambda b: (b, 0, 0)),
            pl.BlockSpec((2 * OD, CD), lambda b: (0, 0)),
            pl.BlockSpec((1, 2 * OD), lambda b: (0, 0)),
            pl.BlockSpec((OD, D), lambda b: (0, 0)),
            pl.BlockSpec((1, OD), lambda b: (0, 0)),
        ],
        out_specs=pl.BlockSpec((BPG * N, OD), lambda b: (b, 0)),
        out_shape=jax.ShapeDtypeStruct((B * N, OD), jnp.float32),
    )(node_feats, cond_feats, W_cond, b_cond.reshape(1, 2 * OD), W_film,
      b_film.reshape(1, OD))


def _make_sc_gather(n_nodes, OD, K, E):
    CE = 128
    CN = CE // K
    NCH = OD // _LANES
    n_chunks = E // CE
    assert n_chunks * CE == E
    BASE = n_chunks // _NW
    assert BASE % 2 == 0
    HALF = (BASE // 2 + 7) // 8 * 8
    REM = n_chunks - _NW * BASE
    assert REM % 2 == 0
    NXW = REM // 2
    CAP = BASE + (2 if REM else 0)
    mesh = plsc.VectorSubcoreMesh(core_axis_name="c", subcore_axis_name="s")

    @functools.partial(
        pl.kernel,
        out_type=jax.ShapeDtypeStruct((n_nodes, OD), jnp.float32),
        mesh=mesh,
        scratch_types=[
            pltpu.VMEM((CAP * CE,), jnp.int32),
            pltpu.VMEM((CAP * CE,), jnp.float32),
            pltpu.VMEM((CAP * CE,), jnp.float32),
            pltpu.VMEM((4, CE, OD), jnp.float32),
            pltpu.VMEM((HALF * CN, OD), jnp.float32),
            pltpu.SemaphoreType.DMA((4,)),
        ],
    )
    def sc_gather(tbl_hbm, idx_hbm, w_hbm, p_hbm, out_hbm, idx_v, w_v, p_v,
                  rows_v, out_v, sem):
        wid = lax.axis_index("s") * _NC + lax.axis_index("c")
        start = wid * BASE + 2 * jnp.minimum(wid, NXW)
        ebase = start * CE
        has_extra = wid < NXW
        pltpu.sync_copy(idx_hbm.at[pl.ds(ebase, BASE * CE)],
                        idx_v.at[pl.ds(0, BASE * CE)])
        pltpu.sync_copy(w_hbm.at[pl.ds(ebase, BASE * CE)],
                        w_v.at[pl.ds(0, BASE * CE)])
        pltpu.sync_copy(p_hbm.at[pl.ds(ebase, BASE * CE)],
                        p_v.at[pl.ds(0, BASE * CE)])

        @pl.when(has_extra)
        def _():
            eb2 = ebase + BASE * CE
            pltpu.sync_copy(idx_hbm.at[pl.ds(eb2, 2 * CE)],
                            idx_v.at[pl.ds(BASE * CE, 2 * CE)])
            pltpu.sync_copy(w_hbm.at[pl.ds(eb2, 2 * CE)],
                            w_v.at[pl.ds(BASE * CE, 2 * CE)])
            pltpu.sync_copy(p_hbm.at[pl.ds(eb2, 2 * CE)],
                            p_v.at[pl.ds(BASE * CE, 2 * CE)])

        lane_splat = [jnp.full((_LANES,), j, jnp.int32) for j in range(_LANES)]

        def issue(ci, slot):
            pltpu.async_copy(tbl_hbm.at[idx_v.at[pl.ds(ci * CE, CE)]],
                             rows_v.at[slot], sem.at[slot])

        def wait(slot):
            pltpu.make_async_copy(tbl_hbm.at[idx_v.at[pl.ds(0, CE)]],
                                  rows_v.at[slot], sem.at[slot]).wait()

        def compute(ci, slot):
            lrow = (ci - jnp.where(ci >= HALF, HALF, 0)) * CN
            for q in range(CN):
                acc = [jnp.zeros((_LANES,), jnp.float32) for _ in range(NCH)]
                for g in range(K // _LANES):
                    off = ci * CE + q * K + g * _LANES
                    ew = (w_v[pl.ds(off, _LANES)] * p_v[pl.ds(off, _LANES)])
                    for jj in range(_LANES):
                        e = q * K + g * _LANES + jj
                        wb = ew.at[lane_splat[jj]].get(
                            mode='promise_in_bounds')
                        for c in range(NCH):
                            r = rows_v[slot, e, pl.ds(c * _LANES, _LANES)]
                            acc[c] = acc[c] + wb * r
                row = lrow + q
                for c in range(NCH):
                    out_v[row, pl.ds(c * _LANES, _LANES)] = jnp.maximum(
                        acc[c], 0.0)

        n_mine = BASE + jnp.where(has_extra, 2, 0)
        issue(0, 0)
        issue(1, 1)
        issue(2, 2)

        def gbody(i, carry):
            @pl.when(i == HALF)
            def _():
                pltpu.sync_copy(out_v.at[pl.ds(0, HALF * CN)],
                                out_hbm.at[pl.ds(start * CN, HALF * CN)])

            slot = lax.rem(i, 4)
            wait(slot)
            compute(i, slot)
            nci = i + 3

            @pl.when(nci < n_mine)
            def _():
                issue(nci, lax.rem(nci, 4))
            return carry

        lax.fori_loop(0, n_mine, gbody, 0)

        tail_base = (start + HALF) * CN
        @pl.when(has_extra)
        def _():
            pltpu.sync_copy(
                out_v.at[pl.ds(0, (BASE + 2 - HALF) * CN)],
                out_hbm.at[pl.ds(tail_base, (BASE + 2 - HALF) * CN)])

        @pl.when(jnp.logical_not(has_extra))
        def _():
            pltpu.sync_copy(
                out_v.at[pl.ds(0, (BASE - HALF) * CN)],
                out_hbm.at[pl.ds(tail_base, (BASE - HALF) * CN)])

    return sc_gather


def kernel(node_feats, cond_feats, weights, params, coords_j, W_cond, b_cond,
           W_film, b_film):
    B, N, D = node_feats.shape
    K = weights.shape[2]
    OD = W_film.shape[0]
    E = B * N * K

    tbl = _film_call(node_feats, cond_feats, W_cond, b_cond, W_film, b_film)
    sc = _make_sc_gather(B * N, OD, K, E)
    idx = (coords_j if coords_j.dtype == jnp.int32
           else coords_j.astype(jnp.int32))
    out = sc(tbl, idx, weights.reshape(E), params.reshape(E))
    return out.reshape(B, N, OD)


# speedup vs baseline: 1.9177x; 1.2182x over previous
import functools

import jax
import jax.numpy as jnp
from jax import lax
from jax.experimental import pallas as pl
from jax.experimental.pallas import tpu as pltpu
from jax.experimental.pallas import tpu_sc as plsc

_NC = 2
_NS = 16
_NW = _NC * _NS
_LANES = 16


def _film_body(od, nf_ref, cond_ref, wt_ref, pt_ref, Wc_ref, bc_ref, Wf_ref,
               bf_ref, tbl_ref, ew_ref):
    bpg, N, D = nf_ref.shape
    K = wt_ref.shape[1]
    nf = nf_ref[...].reshape(bpg * N, D)
    x = lax.dot_general(nf, Wf_ref[...], (((1,), (1,)), ((), ())),
                        preferred_element_type=jnp.float32)
    x = x + bf_ref[...]
    mu = jnp.mean(x, axis=1, keepdims=True)
    xc = x - mu
    var = jnp.mean(xc * xc, axis=1, keepdims=True)
    xn = xc / jnp.sqrt(var + 1e-5)
    cond = cond_ref[...].reshape(bpg, cond_ref.shape[2])
    gb = lax.dot_general(cond, Wc_ref[...], (((1,), (1,)), ((), ())),
                         preferred_element_type=jnp.float32)
    gb = gb + bc_ref[...]
    gamma = (gb[:, :od] + 1.0)[:, None, :]
    beta = gb[:, od:][:, None, :]
    gfull = jnp.broadcast_to(gamma, (bpg, N, od)).reshape(bpg * N, od)
    bfull = jnp.broadcast_to(beta, (bpg, N, od)).reshape(bpg * N, od)
    tbl_ref[...] = jnp.maximum(gfull * xn + bfull, 0.0)
    rows_per_b = (N * K) // 128
    for bb in range(bpg):
        ew = wt_ref[bb] * pt_ref[bb]
        ewT = ew.T
        ew3 = ewT.reshape(N // 4, 4, K)
        for j in range(4):
            ew_ref[pl.ds(bb * rows_per_b, rows_per_b),
                   pl.ds(j * K, K)] = ew3[:, j, :]


def _film_call(node_feats, cond_feats, w_t, p_t, W_cond, b_cond, W_film,
               b_film):
    B, N, D = node_feats.shape
    K = w_t.shape[1]
    OD = W_film.shape[0]
    CD = W_cond.shape[1]
    G = 1
    BPG = B // G
    RPG = (BPG * N * K) // 128
    return pl.pallas_call(
        functools.partial(_film_body, OD),
        grid=(G,),
        in_specs=[
            pl.BlockSpec((BPG, N, D), lambda b: (b, 0, 0)),
            pl.BlockSpec((BPG, 1, CD), lambda b: (b, 0, 0)),
            pl.BlockSpec((BPG, K, N), lambda b: (b, 0, 0)),
            pl.BlockSpec((BPG, K, N), lambda b: (b, 0, 0)),
            pl.BlockSpec((2 * OD, CD), lambda b: (0, 0)),
            pl.BlockSpec((1, 2 * OD), lambda b: (0, 0)),
            pl.BlockSpec((OD, D), lambda b: (0, 0)),
            pl.BlockSpec((1, OD), lambda b: (0, 0)),
        ],
        out_specs=[
            pl.BlockSpec((BPG * N, OD), lambda b: (b, 0)),
            pl.BlockSpec((RPG, 128), lambda b: (b, 0)),
        ],
        out_shape=[
            jax.ShapeDtypeStruct((B * N, OD), jnp.float32),
            jax.ShapeDtypeStruct(((B * N * K) // 128, 128), jnp.float32),
        ],
    )(node_feats, cond_feats, w_t, p_t, W_cond, b_cond.reshape(1, 2 * OD),
      W_film, b_film.reshape(1, OD))


def _make_sc_gather(n_nodes, OD, K, E):
    CE = 128
    CN = CE // K
    NCH = OD // _LANES
    n_chunks = E // CE
    assert n_chunks * CE == E
    SMALL = (n_chunks // _NW) // 8 * 8
    blocks8 = n_chunks // 8
    NBIG = blocks8 - _NW * (SMALL // 8)
    BIG = SMALL + 8
    LEFT = n_chunks - 8 * blocks8
    NM_LAST = SMALL + LEFT
    assert 0 <= NBIG < _NW
    assert SMALL > 0 and LEFT % 2 == 0
    HALF = (BIG // 2 + 7) // 8 * 8
    assert SMALL > HALF and BIG - HALF <= HALF
    CAP = BIG
    mesh = plsc.VectorSubcoreMesh(core_axis_name="c", subcore_axis_name="s")

    @functools.partial(
        pl.kernel,
        out_type=jax.ShapeDtypeStruct((n_nodes, OD), jnp.float32),
        mesh=mesh,
        scratch_types=[
            pltpu.VMEM((CAP * CE,), jnp.int32),
            pltpu.VMEM((CAP, CE), jnp.float32),
            pltpu.VMEM((5, CE, OD), jnp.float32),
            pltpu.VMEM((HALF * CN, OD), jnp.float32),
            pltpu.SemaphoreType.DMA((5,)),
        ],
    )
    def sc_gather(tbl_hbm, idx_hbm, ew_hbm, out_hbm, idx_v, ew_v, rows_v,
                  out_v, sem):
        wid = lax.axis_index("s") * _NC + lax.axis_index("c")
        is_big = wid < NBIG
        is_last = wid == _NW - 1
        start = jnp.where(is_big, wid * BIG,
                          NBIG * BIG + (wid - NBIG) * SMALL)
        n_mine = jnp.where(is_big, BIG,
                           jnp.where(is_last, NM_LAST, SMALL))
        ebase = start * CE

        pltpu.sync_copy(idx_hbm.at[pl.ds(ebase, SMALL * CE)],
                        idx_v.at[pl.ds(0, SMALL * CE)])
        pltpu.sync_copy(ew_hbm.at[pl.ds(start, SMALL)],
                        ew_v.at[pl.ds(0, SMALL)])

        @pl.when(is_big)
        def _():
            pltpu.sync_copy(idx_hbm.at[pl.ds(ebase + SMALL * CE, 8 * CE)],
                            idx_v.at[pl.ds(SMALL * CE, 8 * CE)])
            pltpu.sync_copy(ew_hbm.at[pl.ds(start + SMALL, 8)],
                            ew_v.at[pl.ds(SMALL, 8)])

        if LEFT:
            @pl.when(is_last)
            def _():
                pltpu.sync_copy(
                    idx_hbm.at[pl.ds(ebase + SMALL * CE, LEFT * CE)],
                    idx_v.at[pl.ds(SMALL * CE, LEFT * CE)])
                pltpu.sync_copy(ew_hbm.at[pl.ds(start + SMALL, LEFT)],
                                ew_v.at[pl.ds(SMALL, LEFT)])

        lane_splat = [jnp.full((_LANES,), j, jnp.int32) for j in range(_LANES)]

        def issue(ci, slot):
            pltpu.async_copy(tbl_hbm.at[idx_v.at[pl.ds(ci * CE, CE)]],
                             rows_v.at[slot], sem.at[slot])

        def wait(slot):
            pltpu.make_async_copy(tbl_hbm.at[idx_v.at[pl.ds(0, CE)]],
                                  rows_v.at[slot], sem.at[slot]).wait()

        def compute(ci, slot):
            lrow = (ci - jnp.where(ci >= HALF, HALF, 0)) * CN
            for q in range(CN):
                acc = [jnp.zeros((_LANES,), jnp.float32) for _ in range(NCH)]
                for g in range(K // _LANES):
                    off = q * K + g * _LANES
                    ew = ew_v[ci, pl.ds(off, _LANES)]
                    for jj in range(_LANES):
                        e = q * K + g * _LANES + jj
                        wb = ew.at[lane_splat[jj]].get(
                            mode='promise_in_bounds')
                        for c in range(NCH):
                            r = rows_v[slot, e, pl.ds(c * _LANES, _LANES)]
                            acc[c] = acc[c] + wb * r
                row = lrow + q
                for c in range(NCH):
                    out_v[row, pl.ds(c * _LANES, _LANES)] = jnp.maximum(
                        acc[c], 0.0)

        issue(0, 0)
        issue(1, 1)
        issue(2, 2)
        issue(3, 3)

        def gbody(i, carry):
            @pl.when(i == HALF)
            def _():
                pltpu.sync_copy(out_v.at[pl.ds(0, HALF * CN)],
                                out_hbm.at[pl.ds(start * CN, HALF * CN)])

            slot = lax.rem(i, 5)
            wait(slot)
            compute(i, slot)
            nci = i + 4

            @pl.when(nci < n_mine)
            def _():
                issue(nci, lax.rem(nci, 5))
            return carry

        lax.fori_loop(0, n_mine, gbody, 0)

        tail_base = (start + HALF) * CN

        @pl.when(is_big)
        def _():
            pltpu.sync_copy(out_v.at[pl.ds(0, (BIG - HALF) * CN)],
                            out_hbm.at[pl.ds(tail_base, (BIG - HALF) * CN)])

        @pl.when(jnp.logical_and(jnp.logical_not(is_big),
                                 jnp.logical_not(is_last)))
        def _():
            pltpu.sync_copy(out_v.at[pl.ds(0, (SMALL - HALF) * CN)],
                            out_hbm.at[pl.ds(tail_base, (SMALL - HALF) * CN)])

        @pl.when(is_last)
        def _():
            pltpu.sync_copy(
                out_v.at[pl.ds(0, (NM_LAST - HALF) * CN)],
                out_hbm.at[pl.ds(tail_base, (NM_LAST - HALF) * CN)])

    return sc_gather


def kernel(node_feats, cond_feats, weights, params, coords_j, W_cond, b_cond,
           W_film, b_film):
    B, N, D = node_feats.shape
    K = weights.shape[2]
    OD = W_film.shape[0]
    E = B * N * K

    w_t = weights.transpose(0, 2, 3, 1).reshape(B, K, N)
    p_t = params.transpose(0, 2, 3, 1).reshape(B, K, N)
    tbl, ew = _film_call(node_feats, cond_feats, w_t, p_t, W_cond, b_cond,
                         W_film, b_film)
    sc = _make_sc_gather(B * N, OD, K, E)
    idx = (coords_j if coords_j.dtype == jnp.int32
           else coords_j.astype(jnp.int32))
    out = sc(tbl, idx, ew)
    return out.reshape(B, N, OD)
